# Initial kernel scaffold; baseline (speedup 1.0000x reference)
import jax
import jax.numpy as jnp
from jax.experimental import pallas as pl


def kernel(x, edge_index, pos, adj, s, W1_rel, b1, W1_root, W2_rel, b2, W2_root, W3_rel, b3, W3_root):
    raise NotImplementedError("write your pallas kernel here")


# speedup vs baseline: 10.8829x; 10.8829x over previous
import jax
import jax.numpy as jnp
from jax.experimental import pallas as pl

_N = 68
_E = 544
_H = 128
_K = 16
_EPS = 1e-15


def _fused_body(edge_ref, x_ref, adj_ref, s_ref,
                w1r_ref, b1_ref, w1s_ref,
                w2r_ref, b2_ref, w2s_ref,
                w3r_ref, b3_ref, w3s_ref,
                out_ref, oadj_ref, loss_ref):
    f32 = jnp.float32
    src = edge_ref[0:1, :]
    dst = edge_ref[1:2, :]
    rows = jax.lax.broadcasted_iota(jnp.int32, (_N, _E), 0)
    dst_oh = (rows == dst).astype(f32)
    src_oh = (rows == src).astype(f32)
    a_mat = jax.lax.dot_general(dst_oh, src_oh, (((1,), (1,)), ((), ())),
                                preferred_element_type=f32)

    def layer(h, wr, b, wroot):
        rel = jax.lax.dot_general(h, wr, (((1,), (1,)), ((), ())),
                                  preferred_element_type=f32)
        agg = jnp.dot(a_mat, rel, preferred_element_type=f32)
        root = jax.lax.dot_general(h, wroot, (((1,), (1,)), ((), ())),
                                   preferred_element_type=f32)
        return jnp.maximum(agg + root + b, 0.0)

    h = layer(x_ref[...], w1r_ref[...], b1_ref[...], w1s_ref[...])
    h = layer(h, w2r_ref[...], b2_ref[...], w2s_ref[...])
    h = layer(h, w3r_ref[...], b3_ref[...], w3s_ref[...])

    s = s_ref[...]
    m = jnp.max(s, axis=1, keepdims=True)
    e = jnp.exp(s - m)
    ssm = e / jnp.sum(e, axis=1, keepdims=True)

    out_ref[...] = jax.lax.dot_general(ssm, h, (((0,), (0,)), ((), ())),
                                       preferred_element_type=f32)
    adj = adj_ref[...]
    sta = jax.lax.dot_general(ssm, adj, (((0,), (0,)), ((), ())),
                              preferred_element_type=f32)
    oadj_ref[...] = jnp.dot(sta, ssm, preferred_element_type=f32)

    sst = jax.lax.dot_general(ssm, ssm, (((1,), (1,)), ((), ())),
                              preferred_element_type=f32)
    link = adj - sst
    ll = jnp.sqrt(jnp.sum(link * link, keepdims=True)) / (_N * _N)
    ent = -jnp.sum(ssm * jnp.log(ssm + _EPS), keepdims=True) / _N
    loss_ref[...] = jnp.concatenate([ll, ent], axis=1)


def kernel(x, edge_index, pos, adj, s,
           W1_rel, b1, W1_root, W2_rel, b2, W2_root, W3_rel, b3, W3_root):
    out, out_adj, losses = pl.pallas_call(
        _fused_body,
        out_shape=[
            jax.ShapeDtypeStruct((_K, _H), jnp.float32),
            jax.ShapeDtypeStruct((_K, _K), jnp.float32),
            jax.ShapeDtypeStruct((1, 2), jnp.float32),
        ],
    )(edge_index, x, adj.reshape(_N, _N), s.reshape(_N, _K),
      W1_rel, b1.reshape(1, _H), W1_root,
      W2_rel, b2.reshape(1, _H), W2_root,
      W3_rel, b3.reshape(1, _H), W3_root)
    return (out.reshape(1, _K, _H), out_adj.reshape(1, _K, _K),
            losses[0, 0], losses[0, 1], pos)
